# baseline (speedup 1.0000x reference)
import functools

import jax
import jax.numpy as jnp
from jax import lax
from jax.experimental import pallas as pl
from jax.experimental.pallas import tpu as pltpu
from jax.experimental.pallas import tpu_sc as plsc

B, C_IN, H, W = 1, 2048, 15, 20
FEAT = 512
SCALE = 16
OH, OW = H * SCALE, W * SCALE
HW = OH * OW
NPIX = H * W
N_VOX = 60 * 36 * 60

N_PAD = 131072
VBP = 1024
VB = 512
NBLK = N_PAD // VB
CB = 128
NCC = FEAT // CB
NW = 32
BLK_PER_W = NBLK // NW
CHU = 4


def _stage1_body(w_ref, f2d_ref, gamma_ref, beta_ref, out_ref):
    x = jnp.dot(w_ref[...], f2d_ref[...], preferred_element_type=jnp.float32)
    mean = jnp.mean(x, axis=1, keepdims=True)
    var = jnp.mean(x * x, axis=1, keepdims=True) - mean * mean
    x = (x - mean) * jax.lax.rsqrt(var + 1e-5)
    x = x * gamma_ref[...] + beta_ref[...]
    out_ref[...] = jnp.maximum(x, 0.0)


def _prep_body(idx_ref, *out_ref):
    v = idx_ref[...]
    valid = v < HW
    vc = jnp.where(valid, v, 0)
    py = vc // OW
    px = vc - py * OW
    fy = py.astype(jnp.float32) * (float(H - 1) / (OH - 1))
    fx = px.astype(jnp.float32) * (float(W - 1) / (OW - 1))
    y0 = jnp.floor(fy)
    x0 = jnp.floor(fx)
    dy = fy - y0
    dx = fx - x0
    y0i = y0.astype(jnp.int32)
    x0i = x0.astype(jnp.int32)
    y1i = jnp.minimum(y0i + 1, H - 1)
    x1i = jnp.minimum(x0i + 1, W - 1)
    p00 = y0i * W + x0i
    p01 = y0i * W + x1i
    p10 = y1i * W + x0i
    p11 = y1i * W + x1i
    vf = jnp.where(valid, 1.0, 0.0)
    w00 = (1.0 - dy) * (1.0 - dx) * vf
    w01 = (1.0 - dy) * dx * vf
    w10 = dy * (1.0 - dx) * vf
    w11 = dy * dx * vf
    for r, val in enumerate([p00, p01, p10, p11, w00, w01, w10, w11]):
        out_ref[r][...] = val


def _sc_gather(tbl_hbm, p00_h, p01_h, p10_h, p11_h, w00_h, w01_h, w10_h,
               w11_h, out_hbm, tbl_v, pidx_v, wts_v, out_v, sem):
    wid = lax.axis_index("s") * 2 + lax.axis_index("c")
    p_hs = (p00_h, p01_h, p10_h, p11_h)
    w_hs = (w00_h, w01_h, w10_h, w11_h)

    def cc_body(cc, carry):
        pltpu.sync_copy(tbl_hbm.at[pl.ds(cc * CB, CB), :], tbl_v)

        def blk_body(i, carry2):
            b = wid + i * NW
            if True:
                vbase = b * VB
                cps = [
                    pltpu.async_copy(
                        h.at[pl.ds(vbase, VB)], pidx_v.at[r], sem)
                    for r, h in enumerate(p_hs)
                ] + [
                    pltpu.async_copy(
                        h.at[pl.ds(vbase, VB)], wts_v.at[r], sem)
                    for r, h in enumerate(w_hs)
                ]
                for cp in cps:
                    cp.wait()
                for g in range(VB // 16):
                    s = pl.ds(g * 16, 16)
                    p00 = pidx_v[0, s]
                    p01 = pidx_v[1, s]
                    p10 = pidx_v[2, s]
                    p11 = pidx_v[3, s]
                    w00 = wts_v[0, s]
                    w01 = wts_v[1, s]
                    w10 = wts_v[2, s]
                    w11 = wts_v[3, s]

                    def ch_body(ci, carry3):
                        for u in range(CHU):
                            c = ci * CHU + u
                            cvec = jnp.full((16,), c, jnp.int32)
                            acc = (w00 * plsc.load_gather(tbl_v, [cvec, p00])
                                   + w01 * plsc.load_gather(tbl_v, [cvec, p01])
                                   + w10 * plsc.load_gather(tbl_v, [cvec, p10])
                                   + w11 * plsc.load_gather(tbl_v, [cvec, p11]))
                            out_v[c, s] = acc
                        return carry3

                    lax.fori_loop(0, CB // CHU, ch_body, 0)
                pltpu.sync_copy(
                    out_v, out_hbm.at[pl.ds(cc * CB, CB), pl.ds(vbase, VB)])
            return carry2

        lax.fori_loop(0, BLK_PER_W, blk_body, 0)
        return carry

    lax.fori_loop(0, NCC, cc_body, 0)


def kernel(feature2d, depth_mapping_3d, conv_w, bn_gamma, bn_beta):
    f2d = feature2d.reshape(C_IN, NPIX)
    idx = depth_mapping_3d.reshape(N_VOX).astype(jnp.int32)
    idx = jnp.concatenate(
        [idx, jnp.full((N_PAD - N_VOX,), HW, jnp.int32)])
    tbl = pl.pallas_call(
        _stage1_body,
        out_shape=jax.ShapeDtypeStruct((FEAT, NPIX), jnp.float32),
    )(conv_w, f2d, bn_gamma.reshape(FEAT, 1), bn_beta.reshape(FEAT, 1))

    pw = pl.pallas_call(
        _prep_body,
        grid=(N_PAD // VBP,),
        in_specs=[pl.BlockSpec((VBP,), lambda i: (i,))],
        out_specs=[pl.BlockSpec((VBP,), lambda i: (i,))] * 8,
        out_shape=[jax.ShapeDtypeStruct((N_PAD,), jnp.int32)] * 4
        + [jax.ShapeDtypeStruct((N_PAD,), jnp.float32)] * 4,
    )(idx)

    mesh = plsc.VectorSubcoreMesh(core_axis_name="c", subcore_axis_name="s")
    sc = functools.partial(
        pl.kernel,
        mesh=mesh,
        out_type=jax.ShapeDtypeStruct((FEAT, N_PAD), jnp.float32),
        scratch_types=[
            pltpu.VMEM((CB, NPIX), jnp.float32),
            pltpu.VMEM((4, VB), jnp.int32),
            pltpu.VMEM((4, VB), jnp.float32),
            pltpu.VMEM((CB, VB), jnp.float32),
            pltpu.SemaphoreType.DMA,
        ],
        compiler_params=pltpu.CompilerParams(
            use_tc_tiling_on_sc=False, needs_layout_passes=False),
    )(_sc_gather)
    out = sc(tbl, *pw)
    return out[:, :N_VOX].reshape(1, FEAT, 60, 36, 60)


# speedup vs baseline: 1.4910x; 1.4910x over previous
import functools

import jax
import jax.numpy as jnp
from jax import lax
from jax.experimental import pallas as pl
from jax.experimental.pallas import tpu as pltpu
from jax.experimental.pallas import tpu_sc as plsc

B, C_IN, H, W = 1, 2048, 15, 20
FEAT = 512
SCALE = 16
OH, OW = H * SCALE, W * SCALE
HW = OH * OW
NPIX = H * W
N_VOX = 60 * 36 * 60

N_PAD = 131072
NW = 32
VB = 512
NS = 49152
NT = N_PAD - NS
SC_NBLK = NS // VB
BLK_PER_W = SC_NBLK // NW
CB = 128
NCC = FEAT // CB
CHU = 4
VBP = 1024
VBT = 1024


def _stage1_body(w_ref, f2d_ref, gamma_ref, beta_ref, out_ref):
    x = jnp.dot(w_ref[...], f2d_ref[...], preferred_element_type=jnp.float32)
    mean = jnp.mean(x, axis=1, keepdims=True)
    var = jnp.mean(x * x, axis=1, keepdims=True) - mean * mean
    x = (x - mean) * jax.lax.rsqrt(var + 1e-5)
    x = x * gamma_ref[...] + beta_ref[...]
    out_ref[...] = jnp.maximum(x, 0.0)


def _tc_body(idx_ref, tbl_ref, out_ref):
    v = idx_ref[...]
    valid = v < HW
    vc = jnp.where(valid, v, 0)
    py = vc // OW
    px = vc - py * OW
    fy = py.astype(jnp.float32) * (float(H - 1) / (OH - 1))
    fx = px.astype(jnp.float32) * (float(W - 1) / (OW - 1))
    y0 = jnp.floor(fy)
    x0 = jnp.floor(fx)
    dy = fy - y0
    dx = fx - x0
    y0i = y0.astype(jnp.int32)
    x0i = x0.astype(jnp.int32)
    y1i = jnp.minimum(y0i + 1, H - 1)
    x1i = jnp.minimum(x0i + 1, W - 1)
    ry = jax.lax.broadcasted_iota(jnp.int32, (H, VBT), 0)
    rx = jax.lax.broadcasted_iota(jnp.int32, (W, VBT), 0)
    wy = (jnp.where(ry == y0i[None, :], 1.0 - dy[None, :], 0.0)
          + jnp.where(ry == y1i[None, :], dy[None, :], 0.0))
    wx = (jnp.where(rx == x0i[None, :], 1.0 - dx[None, :], 0.0)
          + jnp.where(rx == x1i[None, :], dx[None, :], 0.0))
    wy = wy * jnp.where(valid, 1.0, 0.0)[None, :]
    m = (wy[:, None, :] * wx[None, :, :]).reshape(H * W, VBT)
    out_ref[...] = jnp.dot(tbl_ref[...], m, preferred_element_type=jnp.float32)


def _prep_body(idx_ref, *out_ref):
    v = idx_ref[...]
    valid = v < HW
    vc = jnp.where(valid, v, 0)
    py = vc // OW
    px = vc - py * OW
    fy = py.astype(jnp.float32) * (float(H - 1) / (OH - 1))
    fx = px.astype(jnp.float32) * (float(W - 1) / (OW - 1))
    y0 = jnp.floor(fy)
    x0 = jnp.floor(fx)
    dy = fy - y0
    dx = fx - x0
    y0i = y0.astype(jnp.int32)
    x0i = x0.astype(jnp.int32)
    y1i = jnp.minimum(y0i + 1, H - 1)
    x1i = jnp.minimum(x0i + 1, W - 1)
    p00 = y0i * W + x0i
    p01 = y0i * W + x1i
    p10 = y1i * W + x0i
    p11 = y1i * W + x1i
    vf = jnp.where(valid, 1.0, 0.0)
    w00 = (1.0 - dy) * (1.0 - dx) * vf
    w01 = (1.0 - dy) * dx * vf
    w10 = dy * (1.0 - dx) * vf
    w11 = dy * dx * vf
    for r, val in enumerate([p00, p01, p10, p11, w00, w01, w10, w11]):
        out_ref[r][...] = val


def _sc_gather(tbl_hbm, p00_h, p01_h, p10_h, p11_h, w00_h, w01_h, w10_h,
               w11_h, out_hbm, tbl_v, pidx_v, wts_v, out_v, sem):
    wid = lax.axis_index("s") * 2 + lax.axis_index("c")
    p_hs = (p00_h, p01_h, p10_h, p11_h)
    w_hs = (w00_h, w01_h, w10_h, w11_h)

    def cc_body(cc, carry):
        pltpu.sync_copy(tbl_hbm.at[pl.ds(cc * CB, CB), :], tbl_v)

        def blk_body(i, carry2):
            b = wid + i * NW
            vbase = b * VB
            cps = [
                pltpu.async_copy(
                    h.at[pl.ds(vbase, VB)], pidx_v.at[r], sem)
                for r, h in enumerate(p_hs)
            ] + [
                pltpu.async_copy(
                    h.at[pl.ds(vbase, VB)], wts_v.at[r], sem)
                for r, h in enumerate(w_hs)
            ]
            for cp in cps:
                cp.wait()
            for g in range(VB // 16):
                s = pl.ds(g * 16, 16)
                p00 = pidx_v[0, s]
                p01 = pidx_v[1, s]
                p10 = pidx_v[2, s]
                p11 = pidx_v[3, s]
                w00 = wts_v[0, s]
                w01 = wts_v[1, s]
                w10 = wts_v[2, s]
                w11 = wts_v[3, s]

                def ch_body(ci, carry3):
                    for u in range(CHU):
                        c = ci * CHU + u
                        cvec = jnp.full((16,), c, jnp.int32)
                        acc = (w00 * plsc.load_gather(tbl_v, [cvec, p00])
                               + w01 * plsc.load_gather(tbl_v, [cvec, p01])
                               + w10 * plsc.load_gather(tbl_v, [cvec, p10])
                               + w11 * plsc.load_gather(tbl_v, [cvec, p11]))
                        out_v[c, s] = acc
                    return carry3

                lax.fori_loop(0, CB // CHU, ch_body, 0)
            pltpu.sync_copy(
                out_v, out_hbm.at[pl.ds(cc * CB, CB), pl.ds(vbase, VB)])
            return carry2

        lax.fori_loop(0, BLK_PER_W, blk_body, 0)
        return carry

    lax.fori_loop(0, NCC, cc_body, 0)


def kernel(feature2d, depth_mapping_3d, conv_w, bn_gamma, bn_beta):
    f2d = feature2d.reshape(C_IN, NPIX)
    idx = depth_mapping_3d.reshape(N_VOX).astype(jnp.int32)
    tbl = pl.pallas_call(
        _stage1_body,
        out_shape=jax.ShapeDtypeStruct((FEAT, NPIX), jnp.float32),
    )(conv_w, f2d, bn_gamma.reshape(FEAT, 1), bn_beta.reshape(FEAT, 1))

    idx_sc = jnp.concatenate(
        [idx[NT:], jnp.full((N_PAD - N_VOX,), HW, jnp.int32)])
    pw = pl.pallas_call(
        _prep_body,
        grid=(NS // VBP,),
        in_specs=[pl.BlockSpec((VBP,), lambda i: (i,))],
        out_specs=[pl.BlockSpec((VBP,), lambda i: (i,))] * 8,
        out_shape=[jax.ShapeDtypeStruct((NS,), jnp.int32)] * 4
        + [jax.ShapeDtypeStruct((NS,), jnp.float32)] * 4,
    )(idx_sc)

    mesh = plsc.VectorSubcoreMesh(core_axis_name="c", subcore_axis_name="s")
    sc = functools.partial(
        pl.kernel,
        mesh=mesh,
        out_type=jax.ShapeDtypeStruct((FEAT, NS), jnp.float32),
        scratch_types=[
            pltpu.VMEM((CB, NPIX), jnp.float32),
            pltpu.VMEM((4, VB), jnp.int32),
            pltpu.VMEM((4, VB), jnp.float32),
            pltpu.VMEM((CB, VB), jnp.float32),
            pltpu.SemaphoreType.DMA,
        ],
        compiler_params=pltpu.CompilerParams(
            use_tc_tiling_on_sc=False, needs_layout_passes=False),
    )(_sc_gather)
    out_sc = sc(tbl, *pw)

    out_tc = pl.pallas_call(
        _tc_body,
        grid=(NT // VBT,),
        in_specs=[
            pl.BlockSpec((VBT,), lambda i: (i,)),
            pl.BlockSpec((FEAT, NPIX), lambda i: (0, 0)),
        ],
        out_specs=pl.BlockSpec((FEAT, VBT), lambda i: (0, i)),
        out_shape=jax.ShapeDtypeStruct((FEAT, NT), jnp.float32),
    )(idx[:NT], tbl)

    out = jnp.concatenate([out_tc, out_sc[:, :N_VOX - NT]], axis=1)
    return out.reshape(1, FEAT, 60, 36, 60)


# speedup vs baseline: 1.9884x; 1.3336x over previous
import functools

import jax
import jax.numpy as jnp
from jax import lax
from jax.experimental import pallas as pl
from jax.experimental.pallas import tpu as pltpu
from jax.experimental.pallas import tpu_sc as plsc

B, C_IN, H, W = 1, 2048, 15, 20
FEAT = 512
SCALE = 16
OH, OW = H * SCALE, W * SCALE
HW = OH * OW
NPIX = H * W
NXUP = H * OW
N_VOX = 60 * 36 * 60

NW = 32
CPT = FEAT // NW
VB = 576
NBLK = N_VOX // VB
VBP = 2880


def _stage1_body(w_ref, f2d_ref, gamma_ref, beta_ref, out_ref):
    x = jnp.dot(w_ref[...], f2d_ref[...], preferred_element_type=jnp.float32)
    mean = jnp.mean(x, axis=1, keepdims=True)
    var = jnp.mean(x * x, axis=1, keepdims=True) - mean * mean
    x = (x - mean) * jax.lax.rsqrt(var + 1e-5)
    x = x * gamma_ref[...] + beta_ref[...]
    out_ref[...] = jnp.maximum(x, 0.0)


def _xup_body(tbl_ref, out_ref):
    oxi = lax.broadcasted_iota(jnp.int32, (W, OW), 1)
    j = lax.broadcasted_iota(jnp.int32, (W, OW), 0)
    fx = oxi.astype(jnp.float32) * (float(W - 1) / (OW - 1))
    x0 = jnp.floor(fx)
    dx = fx - x0
    x0i = x0.astype(jnp.int32)
    x1i = jnp.minimum(x0i + 1, W - 1)
    wx = (jnp.where(j == x0i, 1.0 - dx, 0.0)
          + jnp.where(j == x1i, dx, 0.0))
    for y in range(H):
        out_ref[:, y * OW:(y + 1) * OW] = jnp.dot(
            tbl_ref[:, y * W:(y + 1) * W], wx,
            preferred_element_type=jnp.float32)


def _prep_body(idx_ref, c0_ref, c1_ref, w0_ref, w1_ref):
    v = idx_ref[...]
    valid = v < HW
    vc = jnp.where(valid, v, 0)
    py = vc // OW
    px = vc - py * OW
    fy = py.astype(jnp.float32) * (float(H - 1) / (OH - 1))
    y0 = jnp.floor(fy)
    dy = fy - y0
    y0i = y0.astype(jnp.int32)
    y1i = jnp.minimum(y0i + 1, H - 1)
    vf = jnp.where(valid, 1.0, 0.0)
    c0_ref[...] = y0i * OW + px
    c1_ref[...] = y1i * OW + px
    w0_ref[...] = (1.0 - dy) * vf
    w1_ref[...] = dy * vf


def _sc_gather(tblx_hbm, c0_h, c1_h, w0_h, w1_h, out_hbm,
               tbl_v, ci_v, wf_v, out_v, sem):
    wid = lax.axis_index("s") * 2 + lax.axis_index("c")
    rows = wid * CPT
    pltpu.sync_copy(tblx_hbm.at[pl.ds(rows, CPT), :], tbl_v)

    def blk_body(b, carry):
        vbase = b * VB
        cps = [
            pltpu.async_copy(c0_h.at[pl.ds(vbase, VB)], ci_v.at[0], sem),
            pltpu.async_copy(c1_h.at[pl.ds(vbase, VB)], ci_v.at[1], sem),
            pltpu.async_copy(w0_h.at[pl.ds(vbase, VB)], wf_v.at[0], sem),
            pltpu.async_copy(w1_h.at[pl.ds(vbase, VB)], wf_v.at[1], sem),
        ]
        for cp in cps:
            cp.wait()

        def g_body(g, carry2):
            s = pl.ds(g * 16, 16)
            c0 = ci_v[0, s]
            c1 = ci_v[1, s]
            w0 = wf_v[0, s]
            w1 = wf_v[1, s]
            accs = []
            for c in range(CPT):
                cvec = jnp.full((16,), c, jnp.int32)
                accs.append(w0 * plsc.load_gather(tbl_v, [cvec, c0])
                            + w1 * plsc.load_gather(tbl_v, [cvec, c1]))
            for c in range(CPT):
                out_v[c, s] = accs[c]
            return carry2

        lax.fori_loop(0, VB // 16, g_body, 0)
        pltpu.sync_copy(
            out_v, out_hbm.at[pl.ds(rows, CPT), pl.ds(vbase, VB)])
        return carry

    lax.fori_loop(0, NBLK, blk_body, 0)


def kernel(feature2d, depth_mapping_3d, conv_w, bn_gamma, bn_beta):
    f2d = feature2d.reshape(C_IN, NPIX)
    idx = depth_mapping_3d.reshape(N_VOX).astype(jnp.int32)
    tbl = pl.pallas_call(
        _stage1_body,
        out_shape=jax.ShapeDtypeStruct((FEAT, NPIX), jnp.float32),
    )(conv_w, f2d, bn_gamma.reshape(FEAT, 1), bn_beta.reshape(FEAT, 1))

    tblx = pl.pallas_call(
        _xup_body,
        out_shape=jax.ShapeDtypeStruct((FEAT, NXUP), jnp.float32),
    )(tbl)

    c0, c1, w0, w1 = pl.pallas_call(
        _prep_body,
        out_shape=[jax.ShapeDtypeStruct((N_VOX,), jnp.int32)] * 2
        + [jax.ShapeDtypeStruct((N_VOX,), jnp.float32)] * 2,
    )(idx)

    mesh = plsc.VectorSubcoreMesh(core_axis_name="c", subcore_axis_name="s")
    sc = functools.partial(
        pl.kernel,
        mesh=mesh,
        out_type=jax.ShapeDtypeStruct((FEAT, N_VOX), jnp.float32),
        scratch_types=[
            pltpu.VMEM((CPT, NXUP), jnp.float32),
            pltpu.VMEM((2, VB), jnp.int32),
            pltpu.VMEM((2, VB), jnp.float32),
            pltpu.VMEM((CPT, VB), jnp.float32),
            pltpu.SemaphoreType.DMA,
        ],
        compiler_params=pltpu.CompilerParams(
            use_tc_tiling_on_sc=False, needs_layout_passes=False),
    )(_sc_gather)
    out = sc(tblx, c0, c1, w0, w1)
    return out.reshape(1, FEAT, 60, 36, 60)


# speedup vs baseline: 2.3369x; 1.1753x over previous
import functools

import jax
import jax.numpy as jnp
from jax import lax
from jax.experimental import pallas as pl
from jax.experimental.pallas import tpu as pltpu
from jax.experimental.pallas import tpu_sc as plsc

B, C_IN, H, W = 1, 2048, 15, 20
FEAT = 512
SCALE = 16
OH, OW = H * SCALE, W * SCALE
HW = OH * OW
NPIX = H * W
NXUP = H * OW
N_VOX = 60 * 36 * 60

NW = 32
CPT = FEAT // NW
VB = 576
NBLK = N_VOX // VB
VBP = 2880


def _stage1_body(w_ref, f2d_ref, gamma_ref, beta_ref, out_ref):
    x = jnp.dot(w_ref[...], f2d_ref[...], preferred_element_type=jnp.float32)
    mean = jnp.mean(x, axis=1, keepdims=True)
    var = jnp.mean(x * x, axis=1, keepdims=True) - mean * mean
    x = (x - mean) * jax.lax.rsqrt(var + 1e-5)
    x = x * gamma_ref[...] + beta_ref[...]
    out_ref[...] = jnp.maximum(x, 0.0)


def _xup_body(tbl_ref, out_ref):
    oxi = lax.broadcasted_iota(jnp.int32, (W, OW), 1)
    j = lax.broadcasted_iota(jnp.int32, (W, OW), 0)
    fx = oxi.astype(jnp.float32) * (float(W - 1) / (OW - 1))
    x0 = jnp.floor(fx)
    dx = fx - x0
    x0i = x0.astype(jnp.int32)
    x1i = jnp.minimum(x0i + 1, W - 1)
    wx = (jnp.where(j == x0i, 1.0 - dx, 0.0)
          + jnp.where(j == x1i, dx, 0.0))
    for y in range(H):
        out_ref[:, y * OW:(y + 1) * OW] = jnp.dot(
            tbl_ref[:, y * W:(y + 1) * W], wx,
            preferred_element_type=jnp.float32)


def _prep_body(idx_ref, c0_ref, c1_ref, w0_ref, w1_ref):
    v = idx_ref[...]
    valid = v < HW
    vc = jnp.where(valid, v, 0)
    py = vc // OW
    px = vc - py * OW
    fy = py.astype(jnp.float32) * (float(H - 1) / (OH - 1))
    y0 = jnp.floor(fy)
    dy = fy - y0
    y0i = y0.astype(jnp.int32)
    y1i = jnp.minimum(y0i + 1, H - 1)
    vf = jnp.where(valid, 1.0, 0.0)
    c0_ref[...] = y0i * OW + px
    c1_ref[...] = y1i * OW + px
    w0_ref[...] = (1.0 - dy) * vf
    w1_ref[...] = dy * vf


def _sc_gather(tblx_hbm, c0_h, c1_h, w0_h, w1_h, out_hbm,
               tbl_v, ci_v, wf_v, out_v, sem_in, sem_out):
    wid = lax.axis_index("s") * 2 + lax.axis_index("c")
    rows = wid * CPT
    pltpu.sync_copy(tblx_hbm.at[pl.ds(rows, CPT), :], tbl_v)

    def in_copies(b, par):
        vbase = b * VB
        return [
            pltpu.make_async_copy(
                c0_h.at[pl.ds(vbase, VB)], ci_v.at[par, 0], sem_in),
            pltpu.make_async_copy(
                c1_h.at[pl.ds(vbase, VB)], ci_v.at[par, 1], sem_in),
            pltpu.make_async_copy(
                w0_h.at[pl.ds(vbase, VB)], wf_v.at[par, 0], sem_in),
            pltpu.make_async_copy(
                w1_h.at[pl.ds(vbase, VB)], wf_v.at[par, 1], sem_in),
        ]

    def out_copy(b, par):
        return pltpu.make_async_copy(
            out_v.at[par],
            out_hbm.at[pl.ds(rows, CPT), pl.ds(b * VB, VB)], sem_out)

    for cp in in_copies(0, 0):
        cp.start()

    def blk_body(b, carry):
        par = lax.rem(b, 2)
        for cp in in_copies(b, par):
            cp.wait()

        @pl.when(b + 1 < NBLK)
        def _():
            for cp in in_copies(b + 1, 1 - par):
                cp.start()

        @pl.when(b >= 2)
        def _():
            out_copy(b - 2, par).wait()

        def g_body(g, carry2):
            s = pl.ds(g * 16, 16)
            c0 = ci_v[par, 0, s]
            c1 = ci_v[par, 1, s]
            w0 = wf_v[par, 0, s]
            w1 = wf_v[par, 1, s]
            accs = []
            for c in range(CPT):
                cvec = jnp.full((16,), c, jnp.int32)
                accs.append(w0 * plsc.load_gather(tbl_v, [cvec, c0])
                            + w1 * plsc.load_gather(tbl_v, [cvec, c1]))
            for c in range(CPT):
                out_v[par, c, s] = accs[c]
            return carry2

        lax.fori_loop(0, VB // 16, g_body, 0)
        out_copy(b, par).start()
        return carry

    lax.fori_loop(0, NBLK, blk_body, 0)
    out_copy(NBLK - 2, lax.rem(NBLK - 2, 2)).wait()
    out_copy(NBLK - 1, lax.rem(NBLK - 1, 2)).wait()


def kernel(feature2d, depth_mapping_3d, conv_w, bn_gamma, bn_beta):
    f2d = feature2d.reshape(C_IN, NPIX)
    idx = depth_mapping_3d.reshape(N_VOX).astype(jnp.int32)
    tbl = pl.pallas_call(
        _stage1_body,
        out_shape=jax.ShapeDtypeStruct((FEAT, NPIX), jnp.float32),
    )(conv_w, f2d, bn_gamma.reshape(FEAT, 1), bn_beta.reshape(FEAT, 1))

    tblx = pl.pallas_call(
        _xup_body,
        out_shape=jax.ShapeDtypeStruct((FEAT, NXUP), jnp.float32),
    )(tbl)

    c0, c1, w0, w1 = pl.pallas_call(
        _prep_body,
        out_shape=[jax.ShapeDtypeStruct((N_VOX,), jnp.int32)] * 2
        + [jax.ShapeDtypeStruct((N_VOX,), jnp.float32)] * 2,
    )(idx)

    mesh = plsc.VectorSubcoreMesh(core_axis_name="c", subcore_axis_name="s")
    sc = functools.partial(
        pl.kernel,
        mesh=mesh,
        out_type=jax.ShapeDtypeStruct((FEAT, N_VOX), jnp.float32),
        scratch_types=[
            pltpu.VMEM((CPT, NXUP), jnp.float32),
            pltpu.VMEM((2, 2, VB), jnp.int32),
            pltpu.VMEM((2, 2, VB), jnp.float32),
            pltpu.VMEM((2, CPT, VB), jnp.float32),
            pltpu.SemaphoreType.DMA,
            pltpu.SemaphoreType.DMA,
        ],
        compiler_params=pltpu.CompilerParams(
            use_tc_tiling_on_sc=False, needs_layout_passes=False),
    )(_sc_gather)
    out = sc(tblx, c0, c1, w0, w1)
    return out.reshape(1, FEAT, 60, 36, 60)
